# baseline (speedup 1.0000x reference)
import functools

import jax
import jax.numpy as jnp
from jax import lax
from jax.experimental import pallas as pl
from jax.experimental.pallas import tpu as pltpu
from jax.experimental.pallas import tpu_sc as plsc

HIDDEN = 768
EPS = 1e-12
L = 16
NC, NS = 2, 16
NW = NC * NS
G = HIDDEN // L
CH = 32
_DO_COMPUTE = True


def _combine_tables(pos_emb, type_emb):
    maxp, hidden = pos_emb.shape
    tv = type_emb.shape[0]

    def body(p_ref, t_ref, o_ref):
        p = p_ref[...]
        for t in range(tv):
            o_ref[t * maxp:(t + 1) * maxp, :] = p + t_ref[t:t + 1, :]

    return pl.pallas_call(
        body,
        out_shape=jax.ShapeDtypeStruct((tv * maxp, hidden), jnp.float32),
    )(pos_emb, type_emb)


def _rsqrt(x):
    bi = lax.bitcast_convert_type(x, jnp.int32)
    bi = jnp.int32(0x5F3759DF) - lax.shift_right_arithmetic(bi, 1)
    y = lax.bitcast_convert_type(bi, jnp.float32)
    for _ in range(3):
        y = y * (1.5 - 0.5 * x * y * y)
    return y


@functools.lru_cache(maxsize=None)
def _sc_embed(tok, maxp):
    tpw = tok // NW
    nch = tpw // CH
    npair = nch // 2
    mesh = plsc.VectorSubcoreMesh(core_axis_name="c", subcore_axis_name="s")

    @functools.partial(
        pl.kernel,
        out_type=jax.ShapeDtypeStruct((tok, HIDDEN), jnp.float32),
        mesh=mesh,
        compiler_params=pltpu.CompilerParams(needs_layout_passes=False),
        scratch_types=[
            pltpu.VMEM((CH,), jnp.int32),
            pltpu.VMEM((CH,), jnp.int32),
            pltpu.VMEM((CH,), jnp.int32),
            pltpu.VMEM((CH,), jnp.int32),
            pltpu.VMEM((CH,), jnp.int32),
            pltpu.VMEM((CH, HIDDEN), jnp.float32),
            pltpu.VMEM((CH, HIDDEN), jnp.float32),
            pltpu.VMEM((CH, HIDDEN), jnp.float32),
            pltpu.VMEM((CH, HIDDEN), jnp.float32),
            pltpu.VMEM((HIDDEN,), jnp.float32),
            pltpu.VMEM((HIDDEN,), jnp.float32),
            pltpu.SemaphoreType.DMA,
            pltpu.SemaphoreType.DMA,
            pltpu.SemaphoreType.DMA,
            pltpu.SemaphoreType.DMA,
        ],
    )
    def k(wid_hbm, pid_hbm, tid_hbm, wtab_hbm, ctab_hbm, gamma_hbm, beta_hbm,
          out_hbm, widxA, widxB, cidxA, cidxB, ttmp,
          wrowsA, wrowsB, crowsA, crowsB, gamma_v, beta_v,
          semA, semB, osemA, osemB):
        w = lax.axis_index("s") * NC + lax.axis_index("c")
        base = w * tpw
        pltpu.sync_copy(gamma_hbm, gamma_v)
        pltpu.sync_copy(beta_hbm, beta_v)

        def load_idx(tb, widx_v, cidx_v):
            pltpu.sync_copy(wid_hbm.at[pl.ds(tb, CH)], widx_v)
            pltpu.sync_copy(pid_hbm.at[pl.ds(tb, CH)], cidx_v)
            pltpu.sync_copy(tid_hbm.at[pl.ds(tb, CH)], ttmp)
            for i in range(CH // L):
                sl = pl.ds(i * L, L)
                cidx_v[sl] = cidx_v[sl] + ttmp[sl] * maxp

        def start_gathers(widx_v, cidx_v, wrows_v, crows_v, sem):
            pltpu.async_copy(wtab_hbm.at[widx_v], wrows_v, sem)
            pltpu.async_copy(ctab_hbm.at[cidx_v], crows_v, sem)

        def wait_gathers(widx_v, cidx_v, wrows_v, crows_v, sem):
            pltpu.make_async_copy(wtab_hbm.at[widx_v], wrows_v, sem).wait()
            pltpu.make_async_copy(ctab_hbm.at[cidx_v], crows_v, sem).wait()

        def start_out(tb, wrows_v, osem):
            pltpu.async_copy(wrows_v, out_hbm.at[pl.ds(tb, CH)], osem)

        def wait_out(wrows_v, osem):
            pltpu.make_async_copy(
                wrows_v, out_hbm.at[pl.ds(0, CH)], osem).wait()

        def compute(wrows_v, crows_v):
            @plsc.parallel_loop(0, CH, step=2, unroll=2)
            def tokpair(t0):
                t1 = t0 + 1
                z = jnp.zeros((L,), jnp.float32)
                sv0, qv0, sv1, qv1 = z, z, z, z
                for g in range(G):
                    sl = pl.ds(g * L, L)
                    a0 = wrows_v[t0, sl] + crows_v[t0, sl]
                    a1 = wrows_v[t1, sl] + crows_v[t1, sl]
                    wrows_v[t0, sl] = a0
                    wrows_v[t1, sl] = a1
                    sv0 = sv0 + a0
                    qv0 = qv0 + a0 * a0
                    sv1 = sv1 + a1
                    qv1 = qv1 + a1 * a1
                m0 = jnp.sum(sv0) * (1.0 / HIDDEN)
                m1 = jnp.sum(sv1) * (1.0 / HIDDEN)
                y0 = _rsqrt(jnp.sum(qv0) * (1.0 / HIDDEN) - m0 * m0 + EPS)
                y1 = _rsqrt(jnp.sum(qv1) * (1.0 / HIDDEN) - m1 * m1 + EPS)
                mv0 = jnp.full((L,), m0, jnp.float32)
                mv1 = jnp.full((L,), m1, jnp.float32)
                yv0 = jnp.full((L,), y0, jnp.float32)
                yv1 = jnp.full((L,), y1, jnp.float32)
                for g in range(G):
                    sl = pl.ds(g * L, L)
                    gam = gamma_v[sl]
                    bet = beta_v[sl]
                    s0 = wrows_v[t0, sl]
                    s1 = wrows_v[t1, sl]
                    wrows_v[t0, sl] = (s0 - mv0) * yv0 * gam + bet
                    wrows_v[t1, sl] = (s1 - mv1) * yv1 * gam + bet

        load_idx(base, widxA, cidxA)
        start_gathers(widxA, cidxA, wrowsA, crowsA, semA)

        def pair_body(i, _):
            tb0 = base + (2 * i) * CH
            tb1 = tb0 + CH
            load_idx(tb1, widxB, cidxB)

            @pl.when(i > 0)
            def _():
                wait_out(wrowsB, osemB)
            start_gathers(widxB, cidxB, wrowsB, crowsB, semB)

            wait_gathers(widxA, cidxA, wrowsA, crowsA, semA)
            if _DO_COMPUTE:
                compute(wrowsA, crowsA)
            start_out(tb0, wrowsA, osemA)

            @pl.when(i < npair - 1)
            def _():
                load_idx(tb0 + 2 * CH, widxA, cidxA)
                wait_out(wrowsA, osemA)
                start_gathers(widxA, cidxA, wrowsA, crowsA, semA)

            wait_gathers(widxB, cidxB, wrowsB, crowsB, semB)
            if _DO_COMPUTE:
                compute(wrowsB, crowsB)
            start_out(tb1, wrowsB, osemB)
            return 0
        lax.fori_loop(0, npair, pair_body, 0)

        wait_out(wrowsA, osemA)
        wait_out(wrowsB, osemB)

    return k


def kernel(input_ids, token_type_ids, position_ids, word_emb, pos_emb,
           type_emb, gamma, beta):
    b, s = input_ids.shape
    tok = b * s
    maxp = pos_emb.shape[0]
    ctab = _combine_tables(pos_emb, type_emb)
    wid = input_ids.reshape(tok).astype(jnp.int32)
    pid = position_ids.reshape(tok).astype(jnp.int32)
    tid = token_type_ids.reshape(tok).astype(jnp.int32)
    out = _sc_embed(tok, maxp)(wid, pid, tid, word_emb, ctab, gamma, beta)
    return out.reshape(b, s, HIDDEN)


# speedup vs baseline: 4.0392x; 4.0392x over previous
import functools

import jax
import jax.numpy as jnp
from jax import lax
from jax.experimental import pallas as pl
from jax.experimental.pallas import tpu as pltpu
from jax.experimental.pallas import tpu_sc as plsc

HIDDEN = 768
EPS = 1e-12
L = 16
NC, NS = 2, 16
NW = NC * NS
G = HIDDEN // L
CH = 32


def _combine_tables(pos_emb, type_emb):
    maxp, hidden = pos_emb.shape
    tv = type_emb.shape[0]

    def body(p_ref, t_ref, o_ref):
        p = p_ref[...]
        for t in range(tv):
            o_ref[t * maxp:(t + 1) * maxp, :] = p + t_ref[t:t + 1, :]

    return pl.pallas_call(
        body,
        out_shape=jax.ShapeDtypeStruct((tv * maxp, hidden), jnp.float32),
    )(pos_emb, type_emb)


def _layernorm_tc(x, gamma, beta):
    tok = x.shape[0]
    bt = 1024

    def body(x_ref, g_ref, b_ref, o_ref):
        xv = x_ref[...]
        mean = jnp.mean(xv, axis=1, keepdims=True)
        cent = xv - mean
        var = jnp.mean(cent * cent, axis=1, keepdims=True)
        o_ref[...] = cent * lax.rsqrt(var + EPS) * g_ref[...] + b_ref[...]

    return pl.pallas_call(
        body,
        grid=(tok // bt,),
        in_specs=[
            pl.BlockSpec((bt, HIDDEN), lambda i: (i, 0)),
            pl.BlockSpec((1, HIDDEN), lambda i: (0, 0)),
            pl.BlockSpec((1, HIDDEN), lambda i: (0, 0)),
        ],
        out_specs=pl.BlockSpec((bt, HIDDEN), lambda i: (i, 0)),
        out_shape=jax.ShapeDtypeStruct((tok, HIDDEN), jnp.float32),
    )(x, gamma.reshape(1, HIDDEN), beta.reshape(1, HIDDEN))


@functools.lru_cache(maxsize=None)
def _sc_gather_sum(tok, maxp):
    tpw = tok // NW
    nch = tpw // CH
    npair = nch // 2
    mesh = plsc.VectorSubcoreMesh(core_axis_name="c", subcore_axis_name="s")

    @functools.partial(
        pl.kernel,
        out_type=jax.ShapeDtypeStruct((tok, HIDDEN), jnp.float32),
        mesh=mesh,
        compiler_params=pltpu.CompilerParams(needs_layout_passes=False),
        scratch_types=[
            pltpu.VMEM((CH,), jnp.int32),
            pltpu.VMEM((CH,), jnp.int32),
            pltpu.VMEM((CH,), jnp.int32),
            pltpu.VMEM((CH,), jnp.int32),
            pltpu.VMEM((CH,), jnp.int32),
            pltpu.VMEM((CH, HIDDEN), jnp.float32),
            pltpu.VMEM((CH, HIDDEN), jnp.float32),
            pltpu.VMEM((CH, HIDDEN), jnp.float32),
            pltpu.VMEM((CH, HIDDEN), jnp.float32),
            pltpu.SemaphoreType.DMA,
            pltpu.SemaphoreType.DMA,
            pltpu.SemaphoreType.DMA,
            pltpu.SemaphoreType.DMA,
        ],
    )
    def k(wid_hbm, pid_hbm, tid_hbm, wtab_hbm, ctab_hbm,
          out_hbm, widxA, widxB, cidxA, cidxB, ttmp,
          wrowsA, wrowsB, crowsA, crowsB,
          semA, semB, osemA, osemB):
        w = lax.axis_index("s") * NC + lax.axis_index("c")
        base = w * tpw

        def load_idx(tb, widx_v, cidx_v):
            pltpu.sync_copy(wid_hbm.at[pl.ds(tb, CH)], widx_v)
            pltpu.sync_copy(pid_hbm.at[pl.ds(tb, CH)], cidx_v)
            pltpu.sync_copy(tid_hbm.at[pl.ds(tb, CH)], ttmp)
            for i in range(CH // L):
                sl = pl.ds(i * L, L)
                cidx_v[sl] = cidx_v[sl] + ttmp[sl] * maxp

        def start_gathers(widx_v, cidx_v, wrows_v, crows_v, sem):
            pltpu.async_copy(wtab_hbm.at[widx_v], wrows_v, sem)
            pltpu.async_copy(ctab_hbm.at[cidx_v], crows_v, sem)

        def wait_gathers(widx_v, cidx_v, wrows_v, crows_v, sem):
            pltpu.make_async_copy(wtab_hbm.at[widx_v], wrows_v, sem).wait()
            pltpu.make_async_copy(ctab_hbm.at[cidx_v], crows_v, sem).wait()

        def start_out(tb, wrows_v, osem):
            pltpu.async_copy(wrows_v, out_hbm.at[pl.ds(tb, CH)], osem)

        def wait_out(wrows_v, osem):
            pltpu.make_async_copy(
                wrows_v, out_hbm.at[pl.ds(0, CH)], osem).wait()

        def compute(wrows_v, crows_v):
            @plsc.parallel_loop(0, CH, step=2)
            def tokpair(t0):
                t1 = t0 + 1
                for g in range(G):
                    sl = pl.ds(g * L, L)
                    wrows_v[t0, sl] = wrows_v[t0, sl] + crows_v[t0, sl]
                    wrows_v[t1, sl] = wrows_v[t1, sl] + crows_v[t1, sl]

        load_idx(base, widxA, cidxA)
        start_gathers(widxA, cidxA, wrowsA, crowsA, semA)

        def pair_body(i, _):
            tb0 = base + (2 * i) * CH
            tb1 = tb0 + CH
            load_idx(tb1, widxB, cidxB)

            @pl.when(i > 0)
            def _():
                wait_out(wrowsB, osemB)
            start_gathers(widxB, cidxB, wrowsB, crowsB, semB)

            wait_gathers(widxA, cidxA, wrowsA, crowsA, semA)
            compute(wrowsA, crowsA)
            start_out(tb0, wrowsA, osemA)

            @pl.when(i < npair - 1)
            def _():
                load_idx(tb0 + 2 * CH, widxA, cidxA)
                wait_out(wrowsA, osemA)
                start_gathers(widxA, cidxA, wrowsA, crowsA, semA)

            wait_gathers(widxB, cidxB, wrowsB, crowsB, semB)
            compute(wrowsB, crowsB)
            start_out(tb1, wrowsB, osemB)
            return 0
        lax.fori_loop(0, npair, pair_body, 0)

        wait_out(wrowsA, osemA)
        wait_out(wrowsB, osemB)

    return k


def kernel(input_ids, token_type_ids, position_ids, word_emb, pos_emb,
           type_emb, gamma, beta):
    b, s = input_ids.shape
    tok = b * s
    maxp = pos_emb.shape[0]
    ctab = _combine_tables(pos_emb, type_emb)
    wid = input_ids.reshape(tok).astype(jnp.int32)
    pid = position_ids.reshape(tok).astype(jnp.int32)
    tid = token_type_ids.reshape(tok).astype(jnp.int32)
    sums = _sc_gather_sum(tok, maxp)(wid, pid, tid, word_emb, ctab)
    out = _layernorm_tc(sums, gamma, beta)
    return out.reshape(b, s, HIDDEN)


# speedup vs baseline: 4.0881x; 1.0121x over previous
import functools

import jax
import jax.numpy as jnp
from jax import lax
from jax.experimental import pallas as pl
from jax.experimental.pallas import tpu as pltpu
from jax.experimental.pallas import tpu_sc as plsc

HIDDEN = 768
EPS = 1e-12
L = 16
NC, NS = 2, 16
NW = NC * NS
G = HIDDEN // L
CH = 32


def _combine_tables(pos_emb, type_emb):
    maxp, hidden = pos_emb.shape
    tv = type_emb.shape[0]

    def body(p_ref, t_ref, o_ref):
        p = p_ref[...]
        for t in range(tv):
            o_ref[t * maxp:(t + 1) * maxp, :] = p + t_ref[t:t + 1, :]

    return pl.pallas_call(
        body,
        out_shape=jax.ShapeDtypeStruct((tv * maxp, hidden), jnp.float32),
    )(pos_emb, type_emb)


def _layernorm_tc(x, gamma, beta):
    tok = x.shape[0]
    bt = 2048

    def body(x_ref, g_ref, b_ref, o_ref):
        xv = x_ref[...]
        mean = jnp.mean(xv, axis=1, keepdims=True)
        cent = xv - mean
        var = jnp.mean(cent * cent, axis=1, keepdims=True)
        o_ref[...] = cent * lax.rsqrt(var + EPS) * g_ref[...] + b_ref[...]

    return pl.pallas_call(
        body,
        grid=(tok // bt,),
        in_specs=[
            pl.BlockSpec((bt, HIDDEN), lambda i: (i, 0)),
            pl.BlockSpec((1, HIDDEN), lambda i: (0, 0)),
            pl.BlockSpec((1, HIDDEN), lambda i: (0, 0)),
        ],
        out_specs=pl.BlockSpec((bt, HIDDEN), lambda i: (i, 0)),
        out_shape=jax.ShapeDtypeStruct((tok, HIDDEN), jnp.float32),
    )(x, gamma.reshape(1, HIDDEN), beta.reshape(1, HIDDEN))


@functools.lru_cache(maxsize=None)
def _sc_gather_sum(tok, maxp):
    tpw = tok // NW
    nch = tpw // CH
    npair = nch // 2
    mesh = plsc.VectorSubcoreMesh(core_axis_name="c", subcore_axis_name="s")

    @functools.partial(
        pl.kernel,
        out_type=jax.ShapeDtypeStruct((tok, HIDDEN), jnp.float32),
        mesh=mesh,
        compiler_params=pltpu.CompilerParams(needs_layout_passes=False),
        scratch_types=[
            pltpu.VMEM((CH,), jnp.int32),
            pltpu.VMEM((CH,), jnp.int32),
            pltpu.VMEM((CH,), jnp.int32),
            pltpu.VMEM((CH,), jnp.int32),
            pltpu.VMEM((CH,), jnp.int32),
            pltpu.VMEM((CH, HIDDEN), jnp.float32),
            pltpu.VMEM((CH, HIDDEN), jnp.float32),
            pltpu.VMEM((CH, HIDDEN), jnp.float32),
            pltpu.VMEM((CH, HIDDEN), jnp.float32),
            pltpu.SemaphoreType.DMA,
            pltpu.SemaphoreType.DMA,
            pltpu.SemaphoreType.DMA,
            pltpu.SemaphoreType.DMA,
        ],
    )
    def k(wid_hbm, pid_hbm, tid_hbm, wtab_hbm, ctab_hbm,
          out_hbm, widxA, widxB, cidxA, cidxB, ttmp,
          wrowsA, wrowsB, crowsA, crowsB,
          semA, semB, osemA, osemB):
        w = lax.axis_index("s") * NC + lax.axis_index("c")
        base = w * tpw

        def load_idx(tb, widx_v, cidx_v):
            pltpu.sync_copy(wid_hbm.at[pl.ds(tb, CH)], widx_v)
            pltpu.sync_copy(pid_hbm.at[pl.ds(tb, CH)], cidx_v)
            pltpu.sync_copy(tid_hbm.at[pl.ds(tb, CH)], ttmp)
            for i in range(CH // L):
                sl = pl.ds(i * L, L)
                cidx_v[sl] = cidx_v[sl] + ttmp[sl] * maxp

        def start_gathers(widx_v, cidx_v, wrows_v, crows_v, sem):
            pltpu.async_copy(wtab_hbm.at[widx_v], wrows_v, sem)
            pltpu.async_copy(ctab_hbm.at[cidx_v], crows_v, sem)

        def wait_gathers(widx_v, cidx_v, wrows_v, crows_v, sem):
            pltpu.make_async_copy(wtab_hbm.at[widx_v], wrows_v, sem).wait()
            pltpu.make_async_copy(ctab_hbm.at[cidx_v], crows_v, sem).wait()

        def start_out(tb, wrows_v, osem):
            pltpu.async_copy(wrows_v, out_hbm.at[pl.ds(tb, CH)], osem)

        def wait_out(wrows_v, osem):
            pltpu.make_async_copy(
                wrows_v, out_hbm.at[pl.ds(0, CH)], osem).wait()

        def compute(wrows_v, crows_v):
            @plsc.parallel_loop(0, CH, step=2)
            def tokpair(t0):
                t1 = t0 + 1
                for g in range(G):
                    sl = pl.ds(g * L, L)
                    wrows_v[t0, sl] = wrows_v[t0, sl] + crows_v[t0, sl]
                    wrows_v[t1, sl] = wrows_v[t1, sl] + crows_v[t1, sl]

        load_idx(base, widxA, cidxA)
        start_gathers(widxA, cidxA, wrowsA, crowsA, semA)

        def pair_body(i, _):
            tb0 = base + (2 * i) * CH
            tb1 = tb0 + CH
            load_idx(tb1, widxB, cidxB)

            @pl.when(i > 0)
            def _():
                wait_out(wrowsB, osemB)
            start_gathers(widxB, cidxB, wrowsB, crowsB, semB)

            wait_gathers(widxA, cidxA, wrowsA, crowsA, semA)
            compute(wrowsA, crowsA)
            start_out(tb0, wrowsA, osemA)

            @pl.when(i < npair - 1)
            def _():
                load_idx(tb0 + 2 * CH, widxA, cidxA)
                wait_out(wrowsA, osemA)
                start_gathers(widxA, cidxA, wrowsA, crowsA, semA)

            wait_gathers(widxB, cidxB, wrowsB, crowsB, semB)
            compute(wrowsB, crowsB)
            start_out(tb1, wrowsB, osemB)
            return 0
        lax.fori_loop(0, npair, pair_body, 0)

        wait_out(wrowsA, osemA)
        wait_out(wrowsB, osemB)

    return k


def kernel(input_ids, token_type_ids, position_ids, word_emb, pos_emb,
           type_emb, gamma, beta):
    b, s = input_ids.shape
    tok = b * s
    maxp = pos_emb.shape[0]
    ctab = _combine_tables(pos_emb, type_emb)
    wid = input_ids.reshape(tok).astype(jnp.int32)
    pid = position_ids.reshape(tok).astype(jnp.int32)
    tid = token_type_ids.reshape(tok).astype(jnp.int32)
    sums = _sc_gather_sum(tok, maxp)(wid, pid, tid, word_emb, ctab)
    out = _layernorm_tc(sums, gamma, beta)
    return out.reshape(b, s, HIDDEN)
